# baseline (speedup 1.0000x reference)
import functools

import jax
import jax.numpy as jnp
from jax import lax
from jax.experimental import pallas as pl
from jax.experimental.pallas import tpu as pltpu
from jax.experimental.pallas import tpu_sc as plsc

_NUM_CORES = 2
_NUM_SUBCORES = 16
_NUM_WORKERS = _NUM_CORES * _NUM_SUBCORES
_LANES = 16


@functools.lru_cache(maxsize=None)
def _make_kernel(NB, NT, D):
    BB = NB // _NUM_WORKERS
    assert BB == 128 and D == 64 and NT % 2 == 0
    per_w = BB * NT
    BG = BB // _LANES
    mesh = plsc.VectorSubcoreMesh(core_axis_name="c", subcore_axis_name="s")

    @functools.partial(
        pl.kernel,
        mesh=mesh,
        out_type=jax.ShapeDtypeStruct((NT, D, NB), jnp.float32),
        scratch_types=[
            pltpu.VMEM((per_w,), jnp.int32),
            pltpu.VMEM((per_w,), jnp.int32),
            pltpu.VMEM((2, BB), jnp.int32),
            pltpu.VMEM((2, BB, 2 * D), jnp.float32),
            pltpu.VMEM((2, D, BB), jnp.float32),
            pltpu.SemaphoreType.DMA,
            pltpu.SemaphoreType.DMA,
        ],
        compiler_params=pltpu.CompilerParams(
            use_tc_tiling_on_sc=True, needs_layout_passes=False),
    )
    def gather_kernel(idx_hbm, table_hbm, out_hbm,
                      idx_v, idxt_v, stage_v, rows_v, plane_v, gsem, wsem):
        cid = lax.axis_index("c")
        sid = lax.axis_index("s")
        wid = sid * _NUM_CORES + cid
        b0 = wid * BB

        lane = lax.iota(jnp.int32, _LANES)
        lane_nt = lane * NT

        pltpu.sync_copy(idx_hbm.at[pl.ds(wid * per_w, per_w)], idx_v)

        def build_tmajor(t, _):
            for bg in range(BG):
                src = plsc.load_gather(idx_v, [lane_nt + (bg * _LANES * NT + t)])
                idxt_v[pl.ds(t * BB + bg * _LANES, _LANES)] = src
            return _

        lax.fori_loop(0, NT, build_tmajor, 0)

        def fire(t, pb):
            for bg in range(BG):
                v = idxt_v[pl.ds(t * BB + bg * _LANES, _LANES)]
                stage_v[pb, pl.ds(bg * _LANES, _LANES)] = v >> 1
            pltpu.async_copy(
                table_hbm.at[stage_v.at[pb]], rows_v.at[pb], gsem)

        def drain(pb):
            pltpu.make_async_copy(
                table_hbm.at[stage_v.at[pb]], rows_v.at[pb], gsem).wait()

        def transpose(t, pb):
            gath = rows_v.at[pb]
            plane = plane_v.at[pb]
            cols = []
            for bg in range(BG):
                v = idxt_v[pl.ds(t * BB + bg * _LANES, _LANES)]
                cols.append((v & 1) << 6)
            for d in range(D):
                for bg in range(BG):
                    val = plsc.load_gather(
                        gath, [lane + (bg * _LANES), cols[bg] + d])
                    plane[d, pl.ds(bg * _LANES, _LANES)] = val

        def wdesc(t, pb):
            return pltpu.make_async_copy(
                plane_v.at[pb], out_hbm.at[t, :, pl.ds(b0, BB)], wsem)

        fire(0, 0)
        fire(1, 1)

        def outer(g0):
            for pb in range(2):
                t = g0 + pb
                drain(pb)

                @pl.when(t >= 2)
                def _():
                    wdesc(t - 2, pb).wait()

                transpose(t, pb)

                @pl.when(t + 2 < NT)
                def _():
                    fire(t + 2, pb)

                wdesc(t, pb).start()

        pl.loop(0, NT, step=2)(outer)
        wdesc(NT - 2, 0).wait()
        wdesc(NT - 1, 1).wait()

    return gather_kernel


def kernel(x, table):
    NB, NT = x.shape
    V, D = table.shape
    idx = x.reshape(-1).astype(jnp.int32)
    table2 = table.reshape(V // 2, 2 * D)
    out3 = _make_kernel(NB, NT, D)(idx, table2)
    return jnp.transpose(out3, (2, 0, 1))


# speedup vs baseline: 1.1855x; 1.1855x over previous
import functools

import jax
import jax.numpy as jnp
from jax import lax
from jax.experimental import pallas as pl
from jax.experimental.pallas import tpu as pltpu
from jax.experimental.pallas import tpu_sc as plsc

_NUM_CORES = 2
_NUM_SUBCORES = 16
_NUM_WORKERS = _NUM_CORES * _NUM_SUBCORES
_LANES = 16


@functools.lru_cache(maxsize=None)
def _make_kernel(NB, NT, D):
    BB = NB // _NUM_WORKERS
    assert BB == 128 and D == 64 and NT % 2 == 0
    per_w = BB * NT
    BG = BB // _LANES
    mesh = plsc.VectorSubcoreMesh(core_axis_name="c", subcore_axis_name="s")

    @functools.partial(
        pl.kernel,
        mesh=mesh,
        out_type=jax.ShapeDtypeStruct((NT, D, NB), jnp.float32),
        scratch_types=[
            pltpu.VMEM((per_w,), jnp.int32),
            pltpu.VMEM((per_w,), jnp.int32),
            pltpu.VMEM((2, BB), jnp.int32),
            pltpu.VMEM((2, BB, 2 * D), jnp.float32),
            pltpu.VMEM((2, D, BB), jnp.float32),
            pltpu.SemaphoreType.DMA,
            pltpu.SemaphoreType.DMA,
        ],
        compiler_params=pltpu.CompilerParams(
            use_tc_tiling_on_sc=True, needs_layout_passes=False),
    )
    def gather_kernel(idx_hbm, table_hbm, out_hbm,
                      idx_v, idxt_v, stage_v, rows_v, plane_v, gsem, wsem):
        cid = lax.axis_index("c")
        sid = lax.axis_index("s")
        wid = sid * _NUM_CORES + cid
        b0 = wid * BB

        lane = lax.iota(jnp.int32, _LANES)
        lane_nt = lane * NT

        pltpu.sync_copy(idx_hbm.at[pl.ds(wid * per_w, per_w)], idx_v)

        def build_tmajor(t, _):
            for bg in range(BG):
                src = plsc.load_gather(idx_v, [lane_nt + (bg * _LANES * NT + t)])
                idxt_v[pl.ds(t * BB + bg * _LANES, _LANES)] = src
            return _

        lax.fori_loop(0, NT, build_tmajor, 0)

        def fire(t, pb):
            for bg in range(BG):
                v = idxt_v[pl.ds(t * BB + bg * _LANES, _LANES)]
                stage_v[pb, pl.ds(bg * _LANES, _LANES)] = v >> 1
            pltpu.async_copy(
                table_hbm.at[stage_v.at[pb]], rows_v.at[pb], gsem)

        def drain(pb):
            pltpu.make_async_copy(
                table_hbm.at[stage_v.at[pb]], rows_v.at[pb], gsem).wait()

        rows16 = [lane + (bg * _LANES) for bg in range(BG)]

        def transpose(t, pb):
            gath = rows_v.at[pb]
            plane = plane_v.at[pb]
            cols = []
            for bg in range(BG):
                v = idxt_v[pl.ds(t * BB + bg * _LANES, _LANES)]
                cols.append((v & 1) << 6)
            prev = None
            for d in range(D):
                cur = [plsc.load_gather(gath, [rows16[bg], cols[bg] + d])
                       for bg in range(BG)]
                if prev is not None:
                    for bg in range(BG):
                        plane[d - 1, pl.ds(bg * _LANES, _LANES)] = prev[bg]
                prev = cur
            for bg in range(BG):
                plane[D - 1, pl.ds(bg * _LANES, _LANES)] = prev[bg]

        def wdesc(t, pb):
            return pltpu.make_async_copy(
                plane_v.at[pb], out_hbm.at[t, :, pl.ds(b0, BB)], wsem)

        fire(0, 0)
        fire(1, 1)

        def outer(g0):
            for pb in range(2):
                t = g0 + pb
                drain(pb)

                @pl.when(t >= 2)
                def _():
                    wdesc(t - 2, pb).wait()

                transpose(t, pb)

                @pl.when(t + 2 < NT)
                def _():
                    fire(t + 2, pb)

                wdesc(t, pb).start()

        pl.loop(0, NT, step=2)(outer)
        wdesc(NT - 2, 0).wait()
        wdesc(NT - 1, 1).wait()

    return gather_kernel


def kernel(x, table):
    NB, NT = x.shape
    V, D = table.shape
    idx = x.reshape(-1).astype(jnp.int32)
    table2 = table.reshape(V // 2, 2 * D)
    out3 = _make_kernel(NB, NT, D)(idx, table2)
    return jnp.transpose(out3, (2, 0, 1))


# speedup vs baseline: 1.9306x; 1.6285x over previous
import functools

import jax
import jax.numpy as jnp
from jax import lax
from jax.experimental import pallas as pl
from jax.experimental.pallas import tpu as pltpu
from jax.experimental.pallas import tpu_sc as plsc

_NUM_CORES = 2
_NUM_SUBCORES = 16
_NUM_WORKERS = _NUM_CORES * _NUM_SUBCORES


@functools.lru_cache(maxsize=None)
def _make_gather(B, k, s):
    G = k * s
    per_w = B // _NUM_WORKERS
    n_groups = per_w // G
    assert per_w % G == 0 and n_groups % 2 == 0 and s % 8 == 0
    mesh = plsc.VectorSubcoreMesh(core_axis_name="c", subcore_axis_name="s")

    @functools.partial(
        pl.kernel,
        mesh=mesh,
        out_type=jax.ShapeDtypeStruct((B, 128), jnp.float32),
        scratch_types=[
            pltpu.VMEM((per_w,), jnp.int32),
            pltpu.VMEM((2, G, 128), jnp.float32),
            pltpu.SemaphoreType.DMA,
            pltpu.SemaphoreType.DMA,
        ],
        compiler_params=pltpu.CompilerParams(use_tc_tiling_on_sc=True),
    )
    def gather_kernel(idx_hbm, table_hbm, out_hbm, idx_v, rows_v, gsem, wsem):
        cid = lax.axis_index("c")
        sid = lax.axis_index("s")
        wid = sid * _NUM_CORES + cid
        base = wid * per_w

        pltpu.sync_copy(idx_hbm.at[pl.ds(base, per_w)], idx_v)

        def fire_group(g, b):
            for j in range(k):
                pltpu.async_copy(
                    table_hbm.at[idx_v.at[pl.ds(g * G + j * s, s)]],
                    rows_v.at[b].at[pl.ds(j * s, s)], gsem)

        def drain_group(b):
            for j in range(k):
                pltpu.make_async_copy(
                    table_hbm.at[idx_v.at[pl.ds(j * s, s)]],
                    rows_v.at[b].at[pl.ds(j * s, s)], gsem).wait()

        fire_group(0, 0)
        fire_group(1, 1)

        def outer(g0):
            for b in range(2):
                g = g0 + b
                drain_group(b)
                dst = out_hbm.at[pl.ds(base + g * G, G)]
                pltpu.async_copy(rows_v.at[b], dst, wsem)
                pltpu.make_async_copy(rows_v.at[b], dst, wsem).wait()

                @pl.when(g + 2 < n_groups)
                def _():
                    fire_group(g + 2, b)

        pl.loop(0, n_groups, step=2)(outer)

    return gather_kernel


def kernel(x, table):
    orig_shape = x.shape
    D = table.shape[1]
    idx = x.reshape(-1).astype(jnp.int32)
    B = idx.shape[0]
    table_pad = jnp.pad(table, ((0, 0), (0, 128 - D)))
    out = _make_gather(B, 8, 40)(idx, table_pad)
    return out[:, :D].reshape(*orig_shape, D)


# speedup vs baseline: 1.9330x; 1.0012x over previous
import functools

import jax
import jax.numpy as jnp
from jax import lax
from jax.experimental import pallas as pl
from jax.experimental.pallas import tpu as pltpu
from jax.experimental.pallas import tpu_sc as plsc

_NUM_CORES = 2
_NUM_SUBCORES = 16
_NUM_WORKERS = _NUM_CORES * _NUM_SUBCORES


@functools.lru_cache(maxsize=None)
def _make_gather(B, k, s):
    G = k * s
    per_w = B // _NUM_WORKERS
    n_groups = per_w // G
    assert per_w % G == 0 and n_groups % 2 == 0 and s % 8 == 0
    mesh = plsc.VectorSubcoreMesh(core_axis_name="c", subcore_axis_name="s")

    @functools.partial(
        pl.kernel,
        mesh=mesh,
        out_type=jax.ShapeDtypeStruct((B, 128), jnp.float32),
        scratch_types=[
            pltpu.VMEM((per_w,), jnp.int32),
            pltpu.VMEM((2, G, 128), jnp.float32),
            pltpu.SemaphoreType.DMA,
            pltpu.SemaphoreType.DMA,
        ],
        compiler_params=pltpu.CompilerParams(use_tc_tiling_on_sc=True),
    )
    def gather_kernel(idx_hbm, table_hbm, out_hbm, idx_v, rows_v, gsem, wsem):
        cid = lax.axis_index("c")
        sid = lax.axis_index("s")
        wid = sid * _NUM_CORES + cid
        base = wid * per_w

        pltpu.sync_copy(idx_hbm.at[pl.ds(base, per_w)], idx_v)

        def fire_group(g, b):
            for j in range(k):
                pltpu.async_copy(
                    table_hbm.at[idx_v.at[pl.ds(g * G + j * s, s)]],
                    rows_v.at[b].at[pl.ds(j * s, s)], gsem)

        def drain_group(b):
            for j in range(k):
                pltpu.make_async_copy(
                    table_hbm.at[idx_v.at[pl.ds(j * s, s)]],
                    rows_v.at[b].at[pl.ds(j * s, s)], gsem).wait()

        fire_group(0, 0)
        fire_group(1, 1)

        def outer(g0):
            for b in range(2):
                g = g0 + b
                drain_group(b)
                dst = out_hbm.at[pl.ds(base + g * G, G)]
                pltpu.async_copy(rows_v.at[b], dst, wsem)
                pltpu.make_async_copy(rows_v.at[b], dst, wsem).wait()

                @pl.when(g + 2 < n_groups)
                def _():
                    fire_group(g + 2, b)

        pl.loop(0, n_groups, step=2)(outer)

    return gather_kernel


def kernel(x, table):
    orig_shape = x.shape
    D = table.shape[1]
    idx = x.reshape(-1).astype(jnp.int32)
    B = idx.shape[0]
    table_pad = jax.lax.pad(table, jnp.float32(0), ((0, 0, 0), (0, 128 - D, 0)))
    out = _make_gather(B, 8, 40)(idx, table_pad)
    return out[:, :D].reshape(*orig_shape, D)
